# baseline (speedup 1.0000x reference)
import functools

import jax
import jax.numpy as jnp
from jax import lax
from jax.experimental import pallas as pl
from jax.experimental.pallas import tpu as pltpu
from jax.experimental.pallas import tpu_sc as plsc

NX = 24
NY = 24
P = NX * NY
E = 768
B = 64
NC = 2
NS = 16
NW = NC * NS
PPW = P // NW
CH = PPW * E
LANES = 16
NCHUNK = CH // LANES


def _sc_kernel(x_hbm, row_hbm, col_hbm, out_hbm, row_v, col_v, pos_v, buf, sem):
    wid = lax.axis_index("s") * NC + lax.axis_index("c")
    base = wid * CH

    pltpu.sync_copy(row_hbm, row_v)
    pltpu.sync_copy(col_hbm, col_v)

    def pos_row(i, _):
        p = wid * PPW + i
        r = p // NY
        c = p - r * NY

        def pos_chunk(j, _):
            off = i * E + j * LANES
            pos_v[pl.ds(off, LANES)] = (
                row_v[pl.ds(r * E + j * LANES, LANES)]
                + col_v[pl.ds(c * E + j * LANES, LANES)]
            )
            return 0

        lax.fori_loop(0, E // LANES, pos_chunk, 0)
        return 0

    lax.fori_loop(0, PPW, pos_row, 0)

    def batch_body(b, _):
        src = b * (P * E) + base
        pltpu.sync_copy(x_hbm.at[pl.ds(src, CH)], buf)

        def add_chunk(k, _):
            off = k * LANES
            buf[pl.ds(off, LANES)] = buf[pl.ds(off, LANES)] + pos_v[pl.ds(off, LANES)]
            return 0

        lax.fori_loop(0, NCHUNK, add_chunk, 0)
        pltpu.sync_copy(buf, out_hbm.at[pl.ds(src, CH)])
        return 0

    lax.fori_loop(0, B, batch_body, 0)


@jax.jit
def _run(xf, rf, cf):
    mesh = plsc.VectorSubcoreMesh(core_axis_name="c", subcore_axis_name="s")
    return pl.kernel(
        _sc_kernel,
        mesh=mesh,
        out_type=jax.ShapeDtypeStruct((B * P * E,), jnp.float32),
        scratch_types=[
            pltpu.VMEM((NX * E,), jnp.float32),
            pltpu.VMEM((NY * E,), jnp.float32),
            pltpu.VMEM((CH,), jnp.float32),
            pltpu.VMEM((CH,), jnp.float32),
            pltpu.SemaphoreType.DMA,
        ],
    )(xf, rf, cf)


def kernel(x, row_embed, col_embed):
    out = _run(x.reshape(-1), row_embed.reshape(-1), col_embed.reshape(-1))
    return out.reshape(x.shape)


# speedup vs baseline: 1.8304x; 1.8304x over previous
import jax
import jax.numpy as jnp
from jax import lax
from jax.experimental import pallas as pl
from jax.experimental.pallas import tpu as pltpu
from jax.experimental.pallas import tpu_sc as plsc

NX = 24
NY = 24
P = NX * NY
E = 768
B = 64
NC = 2
NS = 16
NW = NC * NS
PPW = P // NW
CH = PPW * E
LANES = 16
NCHUNK = CH // LANES
UNROLL = 8
NBUF = 4


def _sc_kernel(x_hbm, row_hbm, col_hbm, out_hbm,
               row_v, col_v, pos_v,
               buf0, buf1, buf2, buf3,
               isem0, isem1, isem2, isem3,
               osem0, osem1, osem2, osem3):
    bufs = (buf0, buf1, buf2, buf3)
    isems = (isem0, isem1, isem2, isem3)
    osems = (osem0, osem1, osem2, osem3)

    wid = lax.axis_index("s") * NC + lax.axis_index("c")
    base = wid * CH

    def in_copy(b, i):
        src = b * (P * E) + base
        return pltpu.make_async_copy(x_hbm.at[pl.ds(src, CH)], bufs[i], isems[i])

    def out_copy(b, i):
        dst = b * (P * E) + base
        return pltpu.make_async_copy(bufs[i], out_hbm.at[pl.ds(dst, CH)], osems[i])

    for i in range(NBUF):
        in_copy(i, i).start()

    pltpu.sync_copy(row_hbm, row_v)
    pltpu.sync_copy(col_hbm, col_v)

    def pos_row(i, _):
        p = wid * PPW + i
        r = p // NY
        c = p - r * NY

        def pos_chunk(j, _):
            off = i * E + j * LANES
            pos_v[pl.ds(off, LANES)] = (
                row_v[pl.ds(r * E + j * LANES, LANES)]
                + col_v[pl.ds(c * E + j * LANES, LANES)]
            )
            return 0

        lax.fori_loop(0, E // LANES, pos_chunk, 0)
        return 0

    lax.fori_loop(0, PPW, pos_row, 0)

    def round_body(t, _):
        for i in range(NBUF):
            b = NBUF * t + i
            in_copy(b, i).wait()

            def add_body(k, _):
                for j in range(UNROLL):
                    off = (k * UNROLL + j) * LANES
                    plsc.addupdate(bufs[i].at[pl.ds(off, LANES)],
                                   pos_v[pl.ds(off, LANES)])
                return 0

            lax.fori_loop(0, NCHUNK // UNROLL, add_body, 0)
            out_copy(b, i).start()

            j2 = (i + 2) % NBUF
            br = b - 2

            @pl.when(jnp.logical_and(br >= 0, b + 2 < B))
            def _():
                out_copy(br, j2).wait()
                in_copy(b + 2, j2).start()

        return 0

    lax.fori_loop(0, B // NBUF, round_body, 0)

    for i in range(NBUF):
        out_copy(B - NBUF + i, i).wait()


@jax.jit
def _run(xf, rf, cf):
    mesh = plsc.VectorSubcoreMesh(core_axis_name="c", subcore_axis_name="s")
    return pl.kernel(
        _sc_kernel,
        mesh=mesh,
        out_type=jax.ShapeDtypeStruct((B * P * E,), jnp.float32),
        scratch_types=[
            pltpu.VMEM((NX * E,), jnp.float32),
            pltpu.VMEM((NY * E,), jnp.float32),
            pltpu.VMEM((CH,), jnp.float32),
            pltpu.VMEM((CH,), jnp.float32),
            pltpu.VMEM((CH,), jnp.float32),
            pltpu.VMEM((CH,), jnp.float32),
            pltpu.VMEM((CH,), jnp.float32),
            pltpu.SemaphoreType.DMA,
            pltpu.SemaphoreType.DMA,
            pltpu.SemaphoreType.DMA,
            pltpu.SemaphoreType.DMA,
            pltpu.SemaphoreType.DMA,
            pltpu.SemaphoreType.DMA,
            pltpu.SemaphoreType.DMA,
            pltpu.SemaphoreType.DMA,
        ],
    )(xf, rf, cf)


def kernel(x, row_embed, col_embed):
    out = _run(x.reshape(-1), row_embed.reshape(-1), col_embed.reshape(-1))
    return out.reshape(x.shape)
